# Initial kernel scaffold; baseline (speedup 1.0000x reference)
import jax
import jax.numpy as jnp
from jax.experimental import pallas as pl


def kernel(feat, score):
    raise NotImplementedError("write your pallas kernel here")


# speedup vs baseline: 1.6208x; 1.6208x over previous
import jax
import jax.numpy as jnp
from jax.experimental import pallas as pl
from jax.experimental.pallas import tpu as pltpu
from jax.experimental.pallas import tpu_sc as plsc

_B = 16
_N = 8192
_K_NOR = 5
_K_ABN = 10
_K_HARD = 10
_K_TOT = _K_NOR + _K_ABN + _K_HARD
_N_GATHER_PAD = 512
_GATHER_WINDOW = 16


def _topk_body(s_ref, idx_ref, val_ref):
    s = s_ref[...]
    iota = jax.lax.broadcasted_iota(jnp.int32, s.shape, 1)
    neg = jnp.float32(-3.0e38)

    def take_topk(z, k, collect_from=None):
        idx_cols, val_cols = [], []
        for _ in range(k):
            m = jnp.max(z, axis=1, keepdims=True)
            sel_any = z == m
            idx = jnp.min(jnp.where(sel_any, iota, jnp.int32(_N)),
                          axis=1, keepdims=True)
            sel = iota == idx
            if collect_from is None:
                v = m
            else:
                v = jnp.sum(jnp.where(sel, collect_from, jnp.float32(0.0)),
                            axis=1, keepdims=True)
            z = jnp.where(sel, neg, z)
            idx_cols.append(idx)
            val_cols.append(v)
        return idx_cols, val_cols

    i_nor, v_nor = take_topk(-s, _K_NOR)
    i_abn, v_abn = take_topk(s, _K_ABN)
    i_hard, v_hard = take_topk(-jnp.abs(s - 0.5), _K_HARD, collect_from=s)
    v_nor = [-v for v in v_nor]

    idx_all = jnp.concatenate(i_nor + i_abn + i_hard, axis=1)
    val_all = jnp.concatenate(v_nor + v_abn + v_hard, axis=1)
    pad_i = jnp.zeros((_B, 128 - _K_TOT), jnp.int32)
    pad_v = jnp.zeros((_B, 128 - _K_TOT), jnp.float32)
    idx_ref[...] = jnp.concatenate([idx_all, pad_i], axis=1)
    val_ref[...] = jnp.concatenate([val_all, pad_v], axis=1)


def _run_topk(score):
    return pl.pallas_call(
        _topk_body,
        out_shape=[jax.ShapeDtypeStruct((_B, 128), jnp.int32),
                   jax.ShapeDtypeStruct((_B, 128), jnp.float32)],
    )(score)


def _sc_gather(feat2d, idx_rows):
    f_dim = feat2d.shape[1]
    mesh = plsc.VectorSubcoreMesh(core_axis_name="core",
                                  subcore_axis_name="subcore")

    @pl.kernel(out_type=jax.ShapeDtypeStruct((_N_GATHER_PAD, f_dim),
                                             feat2d.dtype),
               mesh=mesh,
               scratch_types=[pltpu.VMEM((128,), jnp.int32),
                              pltpu.VMEM((_GATHER_WINDOW, 1024), jnp.float32),
                              pltpu.SemaphoreType.DMA])
    def knl(x_hbm, i_hbm, o_hbm, idx_vmem, buf_vmem, sem):
        c = jax.lax.axis_index("core")
        s = jax.lax.axis_index("subcore")
        sid = c * 16 + s
        pltpu.async_copy(i_hbm.at[sid], idx_vmem, sem).wait()
        pltpu.sync_copy(x_hbm.at[idx_vmem.at[pl.ds(0, _GATHER_WINDOW)]],
                        buf_vmem)
        pltpu.async_copy(buf_vmem,
                         o_hbm.at[pl.ds(sid * _GATHER_WINDOW, _GATHER_WINDOW)],
                         sem).wait()

    return knl(feat2d, idx_rows)


def kernel(feat, score):
    b, n, f_dim = feat.shape
    out_i, out_v = _run_topk(score)

    idx_nor = out_i[:, 0:_K_NOR]
    val_nor = out_v[:, 0:_K_NOR]
    idx_abn = out_i[:, _K_NOR:_K_NOR + _K_ABN]
    val_abn = out_v[:, _K_NOR:_K_NOR + _K_ABN]
    idx_hard = out_i[:, _K_NOR + _K_ABN:_K_TOT]
    val_hard = out_v[:, _K_NOR + _K_ABN:_K_TOT]

    base = (jnp.arange(b, dtype=jnp.int32) * n)[:, None]
    flat = jnp.concatenate([
        (idx_nor + base).reshape(-1),
        (idx_abn + base).reshape(-1),
        (idx_hard + base).reshape(-1),
        jnp.zeros((_N_GATHER_PAD - b * _K_TOT,), jnp.int32),
    ]).reshape(_N_GATHER_PAD // _GATHER_WINDOW, _GATHER_WINDOW)
    idx_rows = jnp.pad(flat, ((0, 0), (0, 128 - _GATHER_WINDOW)))

    g = _sc_gather(feat.reshape(b * n, f_dim), idx_rows)

    n_nor = b * _K_NOR
    n_abn = b * _K_ABN
    feat_nor = g[:n_nor].reshape(b, _K_NOR, f_dim)
    feat_abn = g[n_nor:n_nor + n_abn].reshape(b, _K_ABN, f_dim)
    feat_hard = g[n_nor + n_abn:n_nor + n_abn + b * _K_HARD].reshape(
        b, _K_HARD, f_dim)

    return (feat_nor, val_nor, idx_nor,
            feat_abn, val_abn,
            feat_hard, val_hard, idx_hard)


# speedup vs baseline: 1.7743x; 1.0947x over previous
import jax
import jax.numpy as jnp
from jax.experimental import pallas as pl
from jax.experimental.pallas import tpu as pltpu
from jax.experimental.pallas import tpu_sc as plsc

_B = 16
_N = 8192
_K_NOR = 5
_K_ABN = 10
_K_HARD = 10
_K_TOT = _K_NOR + _K_ABN + _K_HARD
_W = 16
_N_GATHER = 512


def _topk_body(s_ref, idx_ref, val_ref, frow_ref):
    s = s_ref[...]
    iota = jax.lax.broadcasted_iota(jnp.int32, s.shape, 1)
    neg = jnp.float32(-3.0e38)

    def take_topk(z, k, collect_from=None):
        idx_cols, val_cols = [], []
        for _ in range(k):
            m = jnp.max(z, axis=1, keepdims=True)
            sel_any = z == m
            idx = jnp.min(jnp.where(sel_any, iota, jnp.int32(_N)),
                          axis=1, keepdims=True)
            sel = iota == idx
            if collect_from is None:
                v = m
            else:
                v = jnp.sum(jnp.where(sel, collect_from, jnp.float32(0.0)),
                            axis=1, keepdims=True)
            z = jnp.where(sel, neg, z)
            idx_cols.append(idx)
            val_cols.append(v)
        return idx_cols, val_cols

    i_nor, v_nor = take_topk(-s, _K_NOR)
    i_abn, v_abn = take_topk(s, _K_ABN)
    i_hard, v_hard = take_topk(-jnp.abs(s - 0.5), _K_HARD, collect_from=s)
    v_nor = [-v for v in v_nor]

    idx_all = jnp.concatenate(i_nor + i_abn + i_hard, axis=1)
    val_all = jnp.concatenate(v_nor + v_abn + v_hard, axis=1)
    pad_i = jnp.zeros((_B, 128 - _K_TOT), jnp.int32)
    pad_v = jnp.zeros((_B, 128 - _K_TOT), jnp.float32)
    idx_ref[...] = jnp.concatenate([idx_all, pad_i], axis=1)
    val_ref[...] = jnp.concatenate([val_all, pad_v], axis=1)

    base = jax.lax.broadcasted_iota(jnp.int32, (_B, 1), 0) * _N
    zc = jnp.zeros((_B, 1), jnp.int32)
    row_a = jnp.concatenate(
        [c + base for c in (i_nor + i_abn)] + [zc] * (128 - 15), axis=1)
    row_b = jnp.concatenate(
        [c + base for c in i_hard] + [zc] * (128 - 10), axis=1)
    frow_ref[...] = jnp.concatenate([row_a, row_b], axis=0)


def _run_topk(score):
    return pl.pallas_call(
        _topk_body,
        out_shape=[jax.ShapeDtypeStruct((_B, 128), jnp.int32),
                   jax.ShapeDtypeStruct((_B, 128), jnp.float32),
                   jax.ShapeDtypeStruct((32, 128), jnp.int32)],
    )(score)


def _sc_gather(feat2d, idx_rows):
    f_dim = feat2d.shape[1]
    mesh = plsc.VectorSubcoreMesh(core_axis_name="core",
                                  subcore_axis_name="subcore")

    @pl.kernel(out_type=jax.ShapeDtypeStruct((_N_GATHER, f_dim),
                                             feat2d.dtype),
               mesh=mesh,
               scratch_types=[pltpu.VMEM((128,), jnp.int32),
                              pltpu.VMEM((_W, 1024), jnp.float32),
                              pltpu.SemaphoreType.DMA])
    def knl(x_hbm, i_hbm, o_hbm, idx_vmem, buf_vmem, sem):
        c = jax.lax.axis_index("core")
        s = jax.lax.axis_index("subcore")
        sid = c * 16 + s
        pltpu.async_copy(i_hbm.at[sid], idx_vmem, sem).wait()
        pltpu.sync_copy(x_hbm.at[idx_vmem.at[pl.ds(0, _W)]], buf_vmem)
        pltpu.async_copy(buf_vmem, o_hbm.at[pl.ds(sid * _W, _W)],
                         sem).wait()

    return knl(feat2d, idx_rows)


def kernel(feat, score):
    b, n, f_dim = feat.shape
    out_i, out_v, idx_rows = _run_topk(score)

    idx_nor = out_i[:, 0:_K_NOR]
    val_nor = out_v[:, 0:_K_NOR]
    val_abn = out_v[:, _K_NOR:_K_NOR + _K_ABN]
    idx_hard = out_i[:, _K_NOR + _K_ABN:_K_TOT]
    val_hard = out_v[:, _K_NOR + _K_ABN:_K_TOT]

    g = _sc_gather(feat.reshape(b * n, f_dim), idx_rows)
    g1 = g[:b * _W].reshape(b, _W, f_dim)
    g2 = g[b * _W:].reshape(b, _W, f_dim)

    feat_nor = g1[:, 0:_K_NOR]
    feat_abn = g1[:, _K_NOR:_K_NOR + _K_ABN]
    feat_hard = g2[:, 0:_K_HARD]

    return (feat_nor, val_nor, idx_nor,
            feat_abn, val_abn,
            feat_hard, val_hard, idx_hard)
